# Initial kernel scaffold; baseline (speedup 1.0000x reference)
import jax
import jax.numpy as jnp
from jax.experimental import pallas as pl


def kernel(inputs):
    raise NotImplementedError("write your pallas kernel here")


# speedup vs baseline: 5.3105x; 5.3105x over previous
import functools

import jax
import jax.numpy as jnp
from jax import lax
from jax.experimental import pallas as pl
from jax.experimental.pallas import tpu as pltpu
from jax.experimental.pallas import tpu_sc as plsc

CROP_H = 480
CROP_W = 480
NUM_CROPS = 2

_NUM_CORES = 2
_NUM_SUBCORES = 16
_NW = _NUM_CORES * _NUM_SUBCORES

_R_CHUNK = 240


def _crop_offsets(B, H, W):
    k = jax.random.key(1)
    kh, kw = jax.random.split(k)
    ih = ((H - CROP_H) * jax.random.uniform(kh, (B, NUM_CROPS))).astype(jnp.int32)
    iw = ((W - CROP_W) * jax.random.uniform(kw, (B, NUM_CROPS))).astype(jnp.int32)
    return ih, iw


def kernel(inputs):
    B, C, H, W = inputs.shape
    ih, iw = _crop_offsets(B, H, W)

    P = B * NUM_CROPS * C
    p = jnp.arange(P)
    b_idx = p // (NUM_CROPS * C)
    n_idx = (p // C) % NUM_CROPS
    c_idx = p % C
    row_start = (b_idx * C + c_idx) * H + ih[b_idx, n_idx]
    col_start = iw[b_idx, n_idx]
    planes_per_w = P // _NW
    rs_rep = jnp.broadcast_to(
        row_start.reshape(_NW, planes_per_w, 1).astype(jnp.int32),
        (_NW, planes_per_w, 16),
    )
    cs_rep = jnp.broadcast_to(
        col_start.reshape(_NW, planes_per_w, 1).astype(jnp.int32),
        (_NW, planes_per_w, 16),
    )

    in2d = inputs.reshape(B * C * H, W)
    mesh = plsc.VectorSubcoreMesh(core_axis_name="c", subcore_axis_name="s")

    @functools.partial(
        pl.kernel,
        out_type=jax.ShapeDtypeStruct((P * CROP_H, CROP_W), jnp.float32),
        mesh=mesh,
        compiler_params=pltpu.CompilerParams(
            use_tc_tiling_on_sc=False, needs_layout_passes=False
        ),
        scratch_types=[
            pltpu.VMEM((planes_per_w, 16), jnp.int32),
            pltpu.VMEM((planes_per_w, 16), jnp.int32),
            pltpu.VMEM((_R_CHUNK,), jnp.int32),
            pltpu.VMEM((_R_CHUNK, W), jnp.float32),
            pltpu.SemaphoreType.DMA,
        ],
    )
    def _crop_copy(in_hbm, rs_hbm, cs_hbm, out_hbm, rs_v, cs_v, idx_v, buf, sem):
        wid = lax.axis_index("s") * _NUM_CORES + lax.axis_index("c")
        pltpu.sync_copy(rs_hbm.at[wid], rs_v)
        pltpu.sync_copy(cs_hbm.at[wid], cs_v)
        iota = lax.iota(jnp.int32, 16)
        for slot in range(planes_per_w):
            rs_vec = rs_v[slot]
            cs = jnp.max(cs_v[slot])
            for r0 in range(0, CROP_H, _R_CHUNK):
                for k in range(0, _R_CHUNK, 16):
                    idx_v[pl.ds(k, 16)] = rs_vec + (r0 + k) + iota

                pltpu.async_copy(in_hbm.at[idx_v], buf, sem).wait()

                def _shift_row(i, carry):
                    for k in range(0, CROP_W, 16):
                        buf[i, pl.ds(k, 16)] = buf[i, pl.ds(cs + k, 16)]
                    return carry

                lax.fori_loop(0, _R_CHUNK, _shift_row, 0)

                dst = (wid * planes_per_w + slot) * CROP_H + r0
                dst = pl.multiple_of(dst, 16)
                pltpu.sync_copy(
                    buf.at[:, pl.ds(0, CROP_W)],
                    out_hbm.at[pl.ds(dst, _R_CHUNK)],
                )

    out2d = _crop_copy(in2d, rs_rep, cs_rep)
    return out2d.reshape(B * NUM_CROPS, C, CROP_H, CROP_W)


# speedup vs baseline: 5.7923x; 1.0907x over previous
import functools

import jax
import jax.numpy as jnp
from jax import lax
from jax.experimental import pallas as pl
from jax.experimental.pallas import tpu as pltpu
from jax.experimental.pallas import tpu_sc as plsc

CROP_H = 480
CROP_W = 480
NUM_CROPS = 2

_NUM_CORES = 2
_NUM_SUBCORES = 16
_NW = _NUM_CORES * _NUM_SUBCORES

_R_CHUNK = 96
_CHUNKS_PER_PLANE = CROP_H // _R_CHUNK


def _crop_offsets(B, H, W):
    k = jax.random.key(1)
    kh, kw = jax.random.split(k)
    ih = ((H - CROP_H) * jax.random.uniform(kh, (B, NUM_CROPS))).astype(jnp.int32)
    iw = ((W - CROP_W) * jax.random.uniform(kw, (B, NUM_CROPS))).astype(jnp.int32)
    return ih, iw


def kernel(inputs):
    B, C, H, W = inputs.shape
    ih, iw = _crop_offsets(B, H, W)

    P = B * NUM_CROPS * C
    p = jnp.arange(P)
    b_idx = p // (NUM_CROPS * C)
    n_idx = (p // C) % NUM_CROPS
    c_idx = p % C
    row_start = (b_idx * C + c_idx) * H + ih[b_idx, n_idx]
    col_start = iw[b_idx, n_idx]
    planes_per_w = P // _NW
    n_chunks = planes_per_w * _CHUNKS_PER_PLANE
    rs_rep = jnp.broadcast_to(
        row_start.reshape(_NW, planes_per_w, 1).astype(jnp.int32),
        (_NW, planes_per_w, 16),
    )
    cs_rep = jnp.broadcast_to(
        col_start.reshape(_NW, planes_per_w, 1).astype(jnp.int32),
        (_NW, planes_per_w, 16),
    )

    in2d = inputs.reshape(B * C * H, W)
    mesh = plsc.VectorSubcoreMesh(core_axis_name="c", subcore_axis_name="s")

    @functools.partial(
        pl.kernel,
        out_type=jax.ShapeDtypeStruct((P * CROP_H, CROP_W), jnp.float32),
        mesh=mesh,
        compiler_params=pltpu.CompilerParams(
            use_tc_tiling_on_sc=False, needs_layout_passes=False
        ),
        scratch_types=[
            pltpu.VMEM((planes_per_w, 16), jnp.int32),
            pltpu.VMEM((planes_per_w, 16), jnp.int32),
            pltpu.VMEM((_R_CHUNK,), jnp.int32),
            pltpu.VMEM((_R_CHUNK,), jnp.int32),
            pltpu.VMEM((_R_CHUNK, W), jnp.float32),
            pltpu.VMEM((_R_CHUNK, W), jnp.float32),
            pltpu.SemaphoreType.DMA,
            pltpu.SemaphoreType.DMA,
            pltpu.SemaphoreType.DMA,
            pltpu.SemaphoreType.DMA,
        ],
    )
    def _crop_copy(
        in_hbm, rs_hbm, cs_hbm, out_hbm,
        rs_v, cs_v, idx0, idx1, buf0, buf1, gs0, gs1, ws0, ws1,
    ):
        wid = lax.axis_index("s") * _NUM_CORES + lax.axis_index("c")
        pltpu.sync_copy(rs_hbm.at[wid], rs_v)
        pltpu.sync_copy(cs_hbm.at[wid], cs_v)
        iota = lax.iota(jnp.int32, 16)
        idx = (idx0, idx1)
        buf = (buf0, buf1)
        gsem = (gs0, gs1)
        wsem = (ws0, ws1)

        def build_idx(g):
            slot, ci = divmod(g, _CHUNKS_PER_PLANE)
            rs_vec = rs_v[slot]
            r0 = ci * _R_CHUNK
            for k in range(0, _R_CHUNK, 16):
                idx[g % 2][pl.ds(k, 16)] = rs_vec + (r0 + k) + iota

        def start_gather(g):
            pltpu.async_copy(in_hbm.at[idx[g % 2]], buf[g % 2], gsem[g % 2])

        def wait_gather(g):
            pltpu.make_async_copy(in_hbm.at[idx[g % 2]], buf[g % 2], gsem[g % 2]).wait()

        def write_args(g):
            slot, ci = divmod(g, _CHUNKS_PER_PLANE)
            dst = (wid * planes_per_w + slot) * CROP_H + ci * _R_CHUNK
            dst = pl.multiple_of(dst, 8)
            return (
                buf[g % 2].at[:, pl.ds(0, CROP_W)],
                out_hbm.at[pl.ds(dst, _R_CHUNK)],
                wsem[g % 2],
            )

        def shift(g):
            slot = g // _CHUNKS_PER_PLANE
            cs = jnp.max(cs_v[slot])
            b = buf[g % 2]

            def _row(i, carry):
                for k in range(0, CROP_W, 16):
                    b[i, pl.ds(k, 16)] = b[i, pl.ds(cs + k, 16)]
                return carry

            lax.fori_loop(0, _R_CHUNK, _row, 0)

        build_idx(0)
        start_gather(0)
        for g in range(n_chunks):
            if g + 1 < n_chunks:
                if g >= 1:
                    pltpu.make_async_copy(*write_args(g - 1)).wait()
                build_idx(g + 1)
                start_gather(g + 1)
            wait_gather(g)
            shift(g)
            pltpu.async_copy(*write_args(g))
        pltpu.make_async_copy(*write_args(n_chunks - 2)).wait()
        pltpu.make_async_copy(*write_args(n_chunks - 1)).wait()

    out2d = _crop_copy(in2d, rs_rep, cs_rep)
    return out2d.reshape(B * NUM_CROPS, C, CROP_H, CROP_W)


# speedup vs baseline: 9.3186x; 1.6088x over previous
import functools

import jax
import jax.numpy as jnp
from jax import lax
from jax.experimental import pallas as pl
from jax.experimental.pallas import tpu as pltpu
from jax.experimental.pallas import tpu_sc as plsc

CROP_H = 480
CROP_W = 480
NUM_CROPS = 2

_NUM_CORES = 2
_NUM_SUBCORES = 16
_NW = _NUM_CORES * _NUM_SUBCORES

_R_CHUNK = 80
_CHUNKS_PER_PLANE = CROP_H // _R_CHUNK
_NBUF = 3
_GROUP = 240


def _crop_offsets(B, H, W):
    k = jax.random.key(1)
    kh, kw = jax.random.split(k)
    ih = ((H - CROP_H) * jax.random.uniform(kh, (B, NUM_CROPS))).astype(jnp.int32)
    iw = ((W - CROP_W) * jax.random.uniform(kw, (B, NUM_CROPS))).astype(jnp.int32)
    return ih, iw


def kernel(inputs):
    B, C, H, W = inputs.shape
    ih, iw = _crop_offsets(B, H, W)

    P = B * NUM_CROPS * C
    p = jnp.arange(P)
    b_idx = p // (NUM_CROPS * C)
    n_idx = (p // C) % NUM_CROPS
    c_idx = p % C
    row_start = (b_idx * C + c_idx) * H + ih[b_idx, n_idx]
    col_start = iw[b_idx, n_idx]
    planes_per_w = P // _NW
    n_chunks = planes_per_w * _CHUNKS_PER_PLANE
    rs_rep = jnp.broadcast_to(
        row_start.reshape(_NW, planes_per_w, 1).astype(jnp.int32),
        (_NW, planes_per_w, 16),
    )
    cs_rep = jnp.broadcast_to(
        col_start.reshape(_NW, planes_per_w, 1).astype(jnp.int32),
        (_NW, planes_per_w, 16),
    )

    in2d = inputs.reshape(B * C * H, W)
    mesh = plsc.VectorSubcoreMesh(core_axis_name="c", subcore_axis_name="s")

    @functools.partial(
        pl.kernel,
        out_type=jax.ShapeDtypeStruct((P * CROP_H, CROP_W), jnp.float32),
        mesh=mesh,
        compiler_params=pltpu.CompilerParams(
            use_tc_tiling_on_sc=False, needs_layout_passes=False
        ),
        scratch_types=[
            pltpu.VMEM((planes_per_w, 16), jnp.int32),
            pltpu.VMEM((planes_per_w, 16), jnp.int32),
        ]
        + [pltpu.VMEM((_R_CHUNK,), jnp.int32)] * _NBUF
        + [pltpu.VMEM((_R_CHUNK, W), jnp.float32)] * _NBUF
        + [pltpu.SemaphoreType.DMA] * (2 * _NBUF),
    )
    def _crop_copy(in_hbm, rs_hbm, cs_hbm, out_hbm, rs_v, cs_v, *scratch):
        idx = scratch[0:_NBUF]
        buf = scratch[_NBUF : 2 * _NBUF]
        gsem = scratch[2 * _NBUF : 3 * _NBUF]
        wsem = scratch[3 * _NBUF : 4 * _NBUF]
        wid = lax.axis_index("s") * _NUM_CORES + lax.axis_index("c")
        pltpu.sync_copy(rs_hbm.at[wid], rs_v)
        pltpu.sync_copy(cs_hbm.at[wid], cs_v)
        iota = lax.iota(jnp.int32, 16)

        def build_idx(g):
            slot, ci = divmod(g, _CHUNKS_PER_PLANE)
            rs_vec = rs_v[slot]
            r0 = ci * _R_CHUNK
            for k in range(0, _R_CHUNK, 16):
                idx[g % _NBUF][pl.ds(k, 16)] = rs_vec + (r0 + k) + iota

        def start_gather(g):
            pltpu.async_copy(in_hbm.at[idx[g % _NBUF]], buf[g % _NBUF], gsem[g % _NBUF])

        def wait_gather(g):
            pltpu.make_async_copy(
                in_hbm.at[idx[g % _NBUF]], buf[g % _NBUF], gsem[g % _NBUF]
            ).wait()

        def write_args(g):
            slot, ci = divmod(g, _CHUNKS_PER_PLANE)
            dst = (wid * planes_per_w + slot) * CROP_H + ci * _R_CHUNK
            dst = pl.multiple_of(dst, 8)
            return (
                buf[g % _NBUF].at[:, pl.ds(0, CROP_W)],
                out_hbm.at[pl.ds(dst, _R_CHUNK)],
                wsem[g % _NBUF],
            )

        def shift(g):
            slot = g // _CHUNKS_PER_PLANE
            cs = jnp.max(cs_v[slot])
            b = buf[g % _NBUF]

            def _row(i, carry):
                for k0 in range(0, CROP_W, _GROUP):
                    vals = [
                        b[i, pl.ds(cs + k0 + k, 16)] for k in range(0, _GROUP, 16)
                    ]
                    for k, v in zip(range(0, _GROUP, 16), vals):
                        b[i, pl.ds(k0 + k, 16)] = v
                return carry

            lax.fori_loop(0, _R_CHUNK, _row, 0)

        build_idx(0)
        start_gather(0)
        for g in range(n_chunks):
            if g + 1 < n_chunks:
                if g >= 2:
                    pltpu.make_async_copy(*write_args(g - 2)).wait()
                build_idx(g + 1)
                start_gather(g + 1)
            wait_gather(g)
            shift(g)
            pltpu.async_copy(*write_args(g))
        for j in range(max(0, n_chunks - 3), n_chunks):
            pltpu.make_async_copy(*write_args(j)).wait()

    out2d = _crop_copy(in2d, rs_rep, cs_rep)
    return out2d.reshape(B * NUM_CROPS, C, CROP_H, CROP_W)
